# baseline (speedup 1.0000x reference)
import functools

import jax
import jax.numpy as jnp
import numpy as np
from jax import lax
from jax.experimental import pallas as pl
from jax.experimental.pallas import tpu as pltpu
from jax.experimental.pallas import tpu_sc as plsc

_BATCH = 16384
_N_FIELDS = 26
_EMBED_DIM = 32
_VOCAB_FIELD = 100000
_VOCAB = _N_FIELDS * _VOCAB_FIELD
_N = _BATCH * _N_FIELDS
_NC = 2
_NS = 16
_NW = _NC * _NS
_PER_W = _N // _NW
_IDX_ROWS = _PER_W // 128
_CHUNK = 1024
_GATHERS = _CHUNK // 128
_NCHUNKS = _PER_W // _CHUNK

_SLAB_V = 512
_NSLABS = _VOCAB // _SLAB_V
_SLABS_PER_W = 159
_PAIRS = 80
_TLIN_ROWS = _VOCAB * _EMBED_DIM // 128

_p = np.arange(_PER_W, dtype=np.int64)
_lb = _p // _N_FIELDS
_f = _p % _N_FIELDS
_SRC = ((_f // 8) * 4096 + (_lb // 128) * 1024 + (_f % 8) * 128
        + (_lb % 128)).astype(np.int32).reshape(_IDX_ROWS, 128)
_OFF = (_f * _VOCAB_FIELD).astype(np.int32).reshape(_IDX_ROWS, 128)

_mesh = plsc.VectorSubcoreMesh(core_axis_name="c", subcore_axis_name="s")


@functools.partial(
    pl.kernel,
    out_type=jax.ShapeDtypeStruct((_TLIN_ROWS, 128), jnp.float32),
    mesh=_mesh,
    scratch_types=[
        pltpu.VMEM((_EMBED_DIM, _SLAB_V), jnp.float32),
        pltpu.VMEM((_EMBED_DIM, _SLAB_V), jnp.float32),
        pltpu.VMEM((128, 128), jnp.float32),
        pltpu.VMEM((128, 128), jnp.float32),
        pltpu.SemaphoreType.DMA,
        pltpu.SemaphoreType.DMA,
        pltpu.SemaphoreType.DMA,
        pltpu.SemaphoreType.DMA,
    ],
    compiler_params=pltpu.CompilerParams(
        use_tc_tiling_on_sc=True, needs_layout_passes=False
    ),
)
def _relayout_kernel(tt_hbm, tail_hbm, tlin_hbm, src_a, src_b, dst_a, dst_b,
                     gsem_a, gsem_b, wsem_a, wsem_b):
    wid = lax.axis_index("s") * _NC + lax.axis_index("c")
    base = wid * _SLABS_PER_W
    iota = lax.iota(jnp.int32, 16)

    def _fire_read(sid, src, gsem):
        v0 = pl.multiple_of(sid * _SLAB_V, _SLAB_V)
        pltpu.async_copy(tt_hbm.at[:, pl.ds(v0, _SLAB_V)], src, gsem)

    def _wait_read(src, gsem):
        pltpu.make_async_copy(
            tt_hbm.at[:, pl.ds(0, _SLAB_V)], src, gsem
        ).wait()

    def _transpose(src, dst):
        def _body(d, carry):
            for c in range(8):
                row_idx = iota + (c & 1) * 16
                col_idx = jnp.full((16,), 4 * d + c // 2, jnp.int32)
                v = plsc.load_gather(src, [row_idx, col_idx])
                dst[d, pl.ds(c * 16, 16)] = v
            return carry

        lax.fori_loop(0, 128, _body, 0)

    def _fire_write(sid, dst, wsem):
        pltpu.async_copy(dst, tlin_hbm.at[pl.ds(sid * 128, 128)], wsem)

    def _wait_write(dst, wsem):
        pltpu.make_async_copy(tlin_hbm.at[pl.ds(0, 128)], dst, wsem).wait()

    _fire_read(base, src_a, gsem_a)

    def _pair(k, carry):
        id_a = base + 2 * k
        id_b = jnp.minimum(id_a + 1, _NSLABS - 1)
        id_a2 = jnp.minimum(id_a + 2, _NSLABS - 1)
        id_a = jnp.minimum(id_a, _NSLABS - 1)

        _fire_read(id_b, src_b, gsem_b)
        _wait_read(src_a, gsem_a)

        @pl.when(k > 0)
        def _():
            _wait_write(dst_a, wsem_a)

        _transpose(src_a, dst_a)
        _fire_write(id_a, dst_a, wsem_a)
        _fire_read(id_a2, src_a, gsem_a)

        _wait_read(src_b, gsem_b)

        @pl.when(k > 0)
        def _():
            _wait_write(dst_b, wsem_b)

        _transpose(src_b, dst_b)
        _fire_write(id_b, dst_b, wsem_b)
        return carry

    lax.fori_loop(0, _PAIRS, _pair, 0)
    _wait_read(src_a, gsem_a)
    _wait_write(dst_a, wsem_a)
    _wait_write(dst_b, wsem_b)

    @pl.when(wid == 0)
    def _():
        pltpu.sync_copy(tail_hbm, dst_a.at[pl.ds(0, 16)])
        pltpu.sync_copy(
            dst_a.at[pl.ds(0, 16)], tlin_hbm.at[pl.ds(_TLIN_ROWS - 16, 16)]
        )


@functools.partial(
    pl.kernel,
    out_type=jax.ShapeDtypeStruct((_N, _EMBED_DIM), jnp.float32),
    mesh=_mesh,
    scratch_types=[
        pltpu.VMEM((128, 128), jnp.int32),
        pltpu.VMEM((_IDX_ROWS, 128), jnp.int32),
        pltpu.VMEM((_IDX_ROWS, 128), jnp.int32),
        pltpu.VMEM((_IDX_ROWS, 128), jnp.int32),
        pltpu.VMEM((_CHUNK, _EMBED_DIM), jnp.float32),
        pltpu.SemaphoreType.DMA,
        pltpu.SemaphoreType.DMA,
    ],
    compiler_params=pltpu.CompilerParams(
        use_tc_tiling_on_sc=False, needs_layout_passes=False
    ),
)
def _embed_kernel(x4_hbm, src_hbm, off_hbm, table_hbm, out_hbm,
                  x_raw, m_v, off_v, idx_v, buf_v, ssem, gsem):
    wid = lax.axis_index("s") * _NC + lax.axis_index("c")
    base = wid * _PER_W

    for f_hi in range(4):
        for bh in range(4):
            r0 = (f_hi * 128 + wid * 4 + bh) * 8
            pltpu.async_copy(
                x4_hbm.at[pl.ds(r0, 8)],
                x_raw.at[pl.ds(f_hi * 32 + bh * 8, 8)],
                ssem,
            )
    pltpu.sync_copy(src_hbm, m_v)
    pltpu.sync_copy(off_hbm, off_v)
    pltpu.make_async_copy(x4_hbm.at[pl.ds(0, 128)], x_raw, ssem).wait()

    def _build(r, carry):
        for c in range(8):
            sl = pl.ds(c * 16, 16)
            m = m_v[r, sl]
            v = plsc.load_gather(x_raw, [m >> 7, m & 127])
            idx_v[r, sl] = v + off_v[r, sl]
        return carry

    lax.fori_loop(0, _IDX_ROWS, _build, 0)

    def _chunk(ci, carry):
        for j in range(_GATHERS):
            pltpu.async_copy(
                table_hbm.at[idx_v.at[ci * _GATHERS + j]],
                buf_v.at[pl.ds(j * 128, 128)],
                gsem,
            )
        pltpu.make_async_copy(
            table_hbm.at[pl.ds(0, _CHUNK)], buf_v, gsem
        ).wait()
        pltpu.sync_copy(buf_v, out_hbm.at[pl.ds(base + ci * _CHUNK, _CHUNK)])
        return carry

    lax.fori_loop(0, _NCHUNKS, _chunk, 0)


def kernel(x, embedding_table):
    tail = embedding_table[_VOCAB - 64:].reshape(16, 128)
    t_lin = _relayout_kernel(embedding_table.T, tail)
    t_rm = t_lin.reshape(_VOCAB, _EMBED_DIM)
    y = jnp.pad(x.T, ((0, 32 - _N_FIELDS), (0, 0)))
    x4 = y.reshape(4, 8, 128, 128).transpose(0, 2, 1, 3).reshape(4096, 128)
    out = _embed_kernel(x4, jnp.asarray(_SRC), jnp.asarray(_OFF), t_rm)
    return out.reshape(_BATCH, _N_FIELDS, _EMBED_DIM)


# speedup vs baseline: 1.7395x; 1.7395x over previous
import functools

import jax
import jax.numpy as jnp
import numpy as np
from jax import lax
from jax.experimental import pallas as pl
from jax.experimental.pallas import tpu as pltpu
from jax.experimental.pallas import tpu_sc as plsc

_BATCH = 16384
_N_FIELDS = 26
_EMBED_DIM = 32
_VOCAB_FIELD = 100000
_VOCAB = _N_FIELDS * _VOCAB_FIELD
_N = _BATCH * _N_FIELDS
_NC = 2
_NS = 16
_NW = _NC * _NS
_PER_W = _N // _NW
_IDX_ROWS = _PER_W // 128
_CHUNK = 1024
_GATHERS = _CHUNK // 128
_NCHUNKS = _PER_W // _CHUNK

_SLAB_V = 512
_NSLABS = _VOCAB // _SLAB_V
_SLABS_PER_W = 159
_PAIRS = 80
_TLIN_ROWS = _VOCAB * _EMBED_DIM // 128

_p = np.arange(_PER_W, dtype=np.int64)
_lb = _p // _N_FIELDS
_f = _p % _N_FIELDS
_SRC = ((_f // 8) * 4096 + (_lb // 128) * 1024 + (_f % 8) * 128
        + (_lb % 128)).astype(np.int32).reshape(_IDX_ROWS, 128)
_OFF = (_f * _VOCAB_FIELD).astype(np.int32).reshape(_IDX_ROWS, 128)

_mesh = plsc.VectorSubcoreMesh(core_axis_name="c", subcore_axis_name="s")


_TBLK = 8192
_TGRID = -(-_VOCAB // _TBLK)


def _tc_transpose_body(in_ref, out_ref):
    xt = in_ref[...].T.reshape(_TBLK // 4, 4, _EMBED_DIM)
    out_ref[...] = jnp.concatenate([xt[:, q, :] for q in range(4)], axis=1)


_tc_transpose = pl.pallas_call(
    _tc_transpose_body,
    grid=(_TGRID,),
    in_specs=[pl.BlockSpec((_EMBED_DIM, _TBLK), lambda i: (0, i))],
    out_specs=pl.BlockSpec((_TBLK // 4, 128), lambda i: (i, 0)),
    out_shape=jax.ShapeDtypeStruct((_TLIN_ROWS, 128), jnp.float32),
)


@functools.partial(
    pl.kernel,
    out_type=jax.ShapeDtypeStruct((_N, _EMBED_DIM), jnp.float32),
    mesh=_mesh,
    scratch_types=[
        pltpu.VMEM((128, 128), jnp.int32),
        pltpu.VMEM((_IDX_ROWS, 128), jnp.int32),
        pltpu.VMEM((_IDX_ROWS, 128), jnp.int32),
        pltpu.VMEM((_IDX_ROWS, 128), jnp.int32),
        pltpu.VMEM((_CHUNK, _EMBED_DIM), jnp.float32),
        pltpu.SemaphoreType.DMA,
        pltpu.SemaphoreType.DMA,
    ],
    compiler_params=pltpu.CompilerParams(
        use_tc_tiling_on_sc=False, needs_layout_passes=False
    ),
)
def _embed_kernel(x4_hbm, src_hbm, off_hbm, table_hbm, out_hbm,
                  x_raw, m_v, off_v, idx_v, buf_v, ssem, gsem):
    wid = lax.axis_index("s") * _NC + lax.axis_index("c")
    base = wid * _PER_W

    for f_hi in range(4):
        for bh in range(4):
            r0 = (f_hi * 128 + wid * 4 + bh) * 8
            pltpu.async_copy(
                x4_hbm.at[pl.ds(r0, 8)],
                x_raw.at[pl.ds(f_hi * 32 + bh * 8, 8)],
                ssem,
            )
    pltpu.sync_copy(src_hbm, m_v)
    pltpu.sync_copy(off_hbm, off_v)
    pltpu.make_async_copy(x4_hbm.at[pl.ds(0, 128)], x_raw, ssem).wait()

    def _build(r, carry):
        for c in range(8):
            sl = pl.ds(c * 16, 16)
            m = m_v[r, sl]
            v = plsc.load_gather(x_raw, [m >> 7, m & 127])
            idx_v[r, sl] = v + off_v[r, sl]
        return carry

    lax.fori_loop(0, _IDX_ROWS, _build, 0)

    def _chunk(ci, carry):
        for j in range(_GATHERS):
            pltpu.async_copy(
                table_hbm.at[idx_v.at[ci * _GATHERS + j]],
                buf_v.at[pl.ds(j * 128, 128)],
                gsem,
            )
        pltpu.make_async_copy(
            table_hbm.at[pl.ds(0, _CHUNK)], buf_v, gsem
        ).wait()
        pltpu.sync_copy(buf_v, out_hbm.at[pl.ds(base + ci * _CHUNK, _CHUNK)])
        return carry

    lax.fori_loop(0, _NCHUNKS, _chunk, 0)


def kernel(x, embedding_table):
    t_lin = _tc_transpose(embedding_table.T)
    t_rm = t_lin.reshape(_VOCAB, _EMBED_DIM)
    y = jnp.pad(x.T, ((0, 32 - _N_FIELDS), (0, 0)))
    x4 = y.reshape(4, 8, 128, 128).transpose(0, 2, 1, 3).reshape(4096, 128)
    out = _embed_kernel(x4, jnp.asarray(_SRC), jnp.asarray(_OFF), t_rm)
    return out.reshape(_BATCH, _N_FIELDS, _EMBED_DIM)
